# baseline (speedup 1.0000x reference)
import functools

import jax
import jax.numpy as jnp
from jax.experimental import pallas as pl
from jax.experimental.pallas import tpu as pltpu

_MM = (((1,), (0,)), ((), ()))
_CH = 512


def _pass1_kernel(adj_ref, ut_ref, i_ref, su_ref, tit_ref, bm_ref,
                  *, bu, bi_sz, nbu, nbi):
    bi = pl.program_id(0)
    bj = pl.program_id(1)
    nch = bu // _CH

    @pl.when((bi == 0) & (bj == 0))
    def _init():
        su_ref[...] = jnp.zeros_like(su_ref)
        tit_ref[...] = jnp.zeros_like(tit_ref)

    ib = i_ref[...].astype(jnp.bfloat16)
    contrib = None
    for k in range(nch):
        sl = pl.ds(k * _CH, _CH)
        a_f32 = adj_ref[sl, :]
        ab = a_f32.astype(jnp.bfloat16)
        acc = adj_ref[pl.ds(k * _CH, 16), :].astype(jnp.int32)
        for m in range(1, 32):
            acc = acc | jax.lax.shift_left(
                adj_ref[pl.ds(k * _CH + m * 16, 16), :].astype(jnp.int32), m)
        bm_ref[pl.ds(k * 16, 16), :] = acc
        su_ref[:, pl.ds(bj * bi_sz, bi_sz)] += jax.lax.dot_general(
            ut_ref[:, pl.ds(bi * bu + k * _CH, _CH)].astype(jnp.bfloat16),
            ab, _MM, preferred_element_type=jnp.float32)
        part = jax.lax.dot_general(
            ab, ib,
            _MM, preferred_element_type=jnp.float32)
        contrib = part if contrib is None else jnp.concatenate(
            [contrib, part], axis=0)
        if k == nch - 1:
            tit_ref[:, pl.ds(bi * bu, bu)] += contrib.T


def _pass2_kernel(bm_ref, su_ref, tit_ref, hu_ref, hit_ref, shi_ref,
                  *, bu, bi_sz, nbu, nbi, leaky):
    bi = pl.program_id(0)
    bj = pl.program_id(1)
    nch = bu // _CH

    @pl.when((bi == 0) & (bj == 0))
    def _zshi():
        shi_ref[...] = jnp.zeros_like(shi_ref)

    @pl.when(bj == 0)
    def _zhu():
        hu_ref[...] = jnp.zeros_like(hu_ref)

    tu_blk = jnp.concatenate(
        [su_ref[:, pl.ds(bj * bi_sz + k * _CH, _CH)].astype(jnp.bfloat16).T
         for k in range(bi_sz // _CH)], axis=0)

    for k in range(nch):
        words = bm_ref[pl.ds(k * 16, 16), :]
        ab = jnp.concatenate(
            [(jax.lax.shift_right_logical(words, m) & 1)
             for m in range(32)], axis=0).astype(jnp.bfloat16)
        hu_ref[pl.ds(k * _CH, _CH), :] += jax.lax.dot_general(
            ab, tu_blk, _MM, preferred_element_type=jnp.float32)
        shi_ref[:, pl.ds(bj * bi_sz, bi_sz)] += jax.lax.dot_general(
            tit_ref[:, pl.ds(bi * bu + k * _CH, _CH)].astype(jnp.bfloat16),
            ab, _MM, preferred_element_type=jnp.float32)

    @pl.when(bj == nbi - 1)
    def _act_u():
        huv = hu_ref[...]
        hu_ref[...] = jnp.where(huv >= 0, huv, leaky * huv)

    @pl.when((bi == nbu - 1) & (bj == nbi - 1))
    def _act_i():
        hi = shi_ref[...]
        hit_ref[...] = jnp.where(hi >= 0, hi, leaky * hi)


@jax.jit
def kernel(adj, user_emb, item_emb):
    n_users, n_items = adj.shape
    hd = user_emb.shape[1]
    bu = min(n_users, 2048)
    bi_sz = min(n_items, 2048)
    nbu = n_users // bu
    nbi = n_items // bi_sz

    p1 = functools.partial(_pass1_kernel, bu=bu, bi_sz=bi_sz,
                           nbu=nbu, nbi=nbi)
    su, tit, bm = pl.pallas_call(
        p1,
        grid=(nbu, nbi),
        in_specs=[
            pl.BlockSpec((bu, bi_sz), lambda i, j: (i, j)),
            pl.BlockSpec((hd, n_users), lambda i, j: (0, 0)),
            pl.BlockSpec((bi_sz, hd), lambda i, j: (j, 0)),
        ],
        out_specs=[
            pl.BlockSpec((hd, n_items), lambda i, j: (0, 0)),
            pl.BlockSpec((hd, n_users), lambda i, j: (0, 0)),
            pl.BlockSpec((bu // 32, bi_sz), lambda i, j: (i, j)),
        ],
        out_shape=[
            jax.ShapeDtypeStruct((hd, n_items), jnp.float32),
            jax.ShapeDtypeStruct((hd, n_users), jnp.float32),
            jax.ShapeDtypeStruct((n_users // 32, n_items), jnp.int32),
        ],
    )(adj, user_emb.T, item_emb)

    p2 = functools.partial(_pass2_kernel, bu=bu, bi_sz=bi_sz,
                           nbu=nbu, nbi=nbi, leaky=0.5)
    h_u, h_i_t = pl.pallas_call(
        p2,
        grid=(nbu, nbi),
        in_specs=[
            pl.BlockSpec((bu // 32, bi_sz), lambda i, j: (i, j)),
            pl.BlockSpec((hd, n_items), lambda i, j: (0, 0)),
            pl.BlockSpec((hd, n_users), lambda i, j: (0, 0)),
        ],
        out_specs=[
            pl.BlockSpec((bu, hd), lambda i, j: (i, 0)),
            pl.BlockSpec((hd, n_items), lambda i, j: (0, 0)),
        ],
        out_shape=[
            jax.ShapeDtypeStruct((n_users, hd), jnp.float32),
            jax.ShapeDtypeStruct((hd, n_items), jnp.float32),
        ],
        scratch_shapes=[
            pltpu.VMEM((hd, n_items), jnp.float32),
        ],
    )(bm, su, tit)

    user_all = jnp.concatenate([user_emb, h_u, h_u], axis=1)
    h_i = h_i_t.T
    item_all = jnp.concatenate([item_emb, h_i, h_i], axis=1)
    return (user_all, item_all)


# speedup vs baseline: 1.0026x; 1.0026x over previous
import functools

import jax
import jax.numpy as jnp
from jax.experimental import pallas as pl
from jax.experimental.pallas import tpu as pltpu

_MM = (((1,), (0,)), ((), ()))
_CH = 512


def _pass1_kernel(adj_ref, ut_ref, i_ref, su_ref, tit_ref, bm_ref,
                  *, bu, bi_sz, nbu, nbi):
    bi = pl.program_id(0)
    bj = pl.program_id(1)
    nch = bu // _CH

    @pl.when((bi == 0) & (bj == 0))
    def _init():
        su_ref[...] = jnp.zeros_like(su_ref)
        tit_ref[...] = jnp.zeros_like(tit_ref)

    ib = i_ref[...].astype(jnp.bfloat16)
    contrib = None
    for k in range(nch):
        sl = pl.ds(k * _CH, _CH)
        a_f32 = adj_ref[sl, :]
        ab = a_f32.astype(jnp.bfloat16)
        bits = a_f32.reshape(32, 16, bi_sz).astype(jnp.int32)
        sh = jax.lax.broadcasted_iota(jnp.int32, (32, 16, bi_sz), 0)
        bm_ref[pl.ds(k * 16, 16), :] = jnp.sum(
            jax.lax.shift_left(bits, sh), axis=0)
        su_ref[:, pl.ds(bj * bi_sz, bi_sz)] += jax.lax.dot_general(
            ut_ref[:, pl.ds(bi * bu + k * _CH, _CH)].astype(jnp.bfloat16),
            ab, _MM, preferred_element_type=jnp.float32)
        part = jax.lax.dot_general(
            ab, ib,
            _MM, preferred_element_type=jnp.float32)
        contrib = part if contrib is None else jnp.concatenate(
            [contrib, part], axis=0)
        if k == nch - 1:
            tit_ref[:, pl.ds(bi * bu, bu)] += contrib.T


def _pass2_kernel(bm_ref, su_ref, tit_ref, hu_ref, hit_ref, shi_ref,
                  *, bu, bi_sz, nbu, nbi, leaky):
    bi = pl.program_id(0)
    bj = pl.program_id(1)
    nch = bu // _CH

    @pl.when((bi == 0) & (bj == 0))
    def _zshi():
        shi_ref[...] = jnp.zeros_like(shi_ref)

    @pl.when(bj == 0)
    def _zhu():
        hu_ref[...] = jnp.zeros_like(hu_ref)

    tu_blk = jnp.concatenate(
        [su_ref[:, pl.ds(bj * bi_sz + k * _CH, _CH)].astype(jnp.bfloat16).T
         for k in range(bi_sz // _CH)], axis=0)

    for k in range(nch):
        words = bm_ref[pl.ds(k * 16, 16), :]
        ab = jnp.concatenate(
            [(jax.lax.shift_right_logical(words, m) & 1)
             for m in range(32)], axis=0).astype(jnp.bfloat16)
        hu_ref[pl.ds(k * _CH, _CH), :] += jax.lax.dot_general(
            ab, tu_blk, _MM, preferred_element_type=jnp.float32)
        shi_ref[:, pl.ds(bj * bi_sz, bi_sz)] += jax.lax.dot_general(
            tit_ref[:, pl.ds(bi * bu + k * _CH, _CH)].astype(jnp.bfloat16),
            ab, _MM, preferred_element_type=jnp.float32)

    @pl.when(bj == nbi - 1)
    def _act_u():
        huv = hu_ref[...]
        hu_ref[...] = jnp.where(huv >= 0, huv, leaky * huv)

    @pl.when((bi == nbu - 1) & (bj == nbi - 1))
    def _act_i():
        hi = shi_ref[...]
        hit_ref[...] = jnp.where(hi >= 0, hi, leaky * hi)


@jax.jit
def kernel(adj, user_emb, item_emb):
    n_users, n_items = adj.shape
    hd = user_emb.shape[1]
    bu = min(n_users, 2048)
    bi_sz = min(n_items, 2048)
    nbu = n_users // bu
    nbi = n_items // bi_sz

    p1 = functools.partial(_pass1_kernel, bu=bu, bi_sz=bi_sz,
                           nbu=nbu, nbi=nbi)
    su, tit, bm = pl.pallas_call(
        p1,
        grid=(nbu, nbi),
        in_specs=[
            pl.BlockSpec((bu, bi_sz), lambda i, j: (i, j)),
            pl.BlockSpec((hd, n_users), lambda i, j: (0, 0)),
            pl.BlockSpec((bi_sz, hd), lambda i, j: (j, 0)),
        ],
        out_specs=[
            pl.BlockSpec((hd, n_items), lambda i, j: (0, 0)),
            pl.BlockSpec((hd, n_users), lambda i, j: (0, 0)),
            pl.BlockSpec((bu // 32, bi_sz), lambda i, j: (i, j)),
        ],
        out_shape=[
            jax.ShapeDtypeStruct((hd, n_items), jnp.float32),
            jax.ShapeDtypeStruct((hd, n_users), jnp.float32),
            jax.ShapeDtypeStruct((n_users // 32, n_items), jnp.int32),
        ],
    )(adj, user_emb.T, item_emb)

    p2 = functools.partial(_pass2_kernel, bu=bu, bi_sz=bi_sz,
                           nbu=nbu, nbi=nbi, leaky=0.5)
    h_u, h_i_t = pl.pallas_call(
        p2,
        grid=(nbu, nbi),
        in_specs=[
            pl.BlockSpec((bu // 32, bi_sz), lambda i, j: (i, j)),
            pl.BlockSpec((hd, n_items), lambda i, j: (0, 0)),
            pl.BlockSpec((hd, n_users), lambda i, j: (0, 0)),
        ],
        out_specs=[
            pl.BlockSpec((bu, hd), lambda i, j: (i, 0)),
            pl.BlockSpec((hd, n_items), lambda i, j: (0, 0)),
        ],
        out_shape=[
            jax.ShapeDtypeStruct((n_users, hd), jnp.float32),
            jax.ShapeDtypeStruct((hd, n_items), jnp.float32),
        ],
        scratch_shapes=[
            pltpu.VMEM((hd, n_items), jnp.float32),
        ],
    )(bm, su, tit)

    user_all = jnp.concatenate([user_emb, h_u, h_u], axis=1)
    h_i = h_i_t.T
    item_all = jnp.concatenate([item_emb, h_i, h_i], axis=1)
    return (user_all, item_all)


# speedup vs baseline: 1.0027x; 1.0001x over previous
import functools

import jax
import jax.numpy as jnp
from jax.experimental import pallas as pl
from jax.experimental.pallas import tpu as pltpu

_MM = (((1,), (0,)), ((), ()))
_CH = 512


def _pass1_kernel(adj_ref, ut_ref, i_ref, su_ref, tit_ref, bm_ref,
                  *, bu, bi_sz, nbu, nbi):
    bi = pl.program_id(0)
    bj = pl.program_id(1)
    nch = bu // _CH

    @pl.when((bi == 0) & (bj == 0))
    def _init():
        su_ref[...] = jnp.zeros_like(su_ref)
        tit_ref[...] = jnp.zeros_like(tit_ref)

    ib = i_ref[...].astype(jnp.bfloat16)
    contrib = None
    for k in range(nch):
        sl = pl.ds(k * _CH, _CH)
        a_f32 = adj_ref[sl, :]
        ab = a_f32.astype(jnp.bfloat16)
        su_ref[:, pl.ds(bj * bi_sz, bi_sz)] += jax.lax.dot_general(
            ut_ref[:, pl.ds(bi * bu + k * _CH, _CH)].astype(jnp.bfloat16),
            ab, _MM, preferred_element_type=jnp.float32)
        part = jax.lax.dot_general(
            ab, ib,
            _MM, preferred_element_type=jnp.float32)
        bits = a_f32.reshape(32, 16, bi_sz).astype(jnp.int32)
        sh = jax.lax.broadcasted_iota(jnp.int32, (32, 16, bi_sz), 0)
        bm_ref[pl.ds(k * 16, 16), :] = jnp.sum(
            jax.lax.shift_left(bits, sh), axis=0)
        contrib = part if contrib is None else jnp.concatenate(
            [contrib, part], axis=0)
        if k == nch - 1:
            tit_ref[:, pl.ds(bi * bu, bu)] += contrib.T


def _pass2_kernel(bm_ref, su_ref, tit_ref, hu_ref, hit_ref, shi_ref,
                  *, bu, bi_sz, nbu, nbi, leaky):
    bi = pl.program_id(0)
    bj = pl.program_id(1)
    nch = bu // _CH

    @pl.when((bi == 0) & (bj == 0))
    def _zshi():
        shi_ref[...] = jnp.zeros_like(shi_ref)

    @pl.when(bj == 0)
    def _zhu():
        hu_ref[...] = jnp.zeros_like(hu_ref)

    tu_blk = jnp.concatenate(
        [su_ref[:, pl.ds(bj * bi_sz + k * _CH, _CH)].astype(jnp.bfloat16).T
         for k in range(bi_sz // _CH)], axis=0)

    for k in range(nch):
        words = bm_ref[pl.ds(k * 16, 16), :]
        ab = jnp.concatenate(
            [(jax.lax.shift_right_logical(words, m) & 1)
             for m in range(32)], axis=0).astype(jnp.bfloat16)
        hu_ref[pl.ds(k * _CH, _CH), :] += jax.lax.dot_general(
            ab, tu_blk, _MM, preferred_element_type=jnp.float32)
        shi_ref[:, pl.ds(bj * bi_sz, bi_sz)] += jax.lax.dot_general(
            tit_ref[:, pl.ds(bi * bu + k * _CH, _CH)].astype(jnp.bfloat16),
            ab, _MM, preferred_element_type=jnp.float32)

    @pl.when(bj == nbi - 1)
    def _act_u():
        huv = hu_ref[...]
        hu_ref[...] = jnp.where(huv >= 0, huv, leaky * huv)

    @pl.when((bi == nbu - 1) & (bj == nbi - 1))
    def _act_i():
        hi = shi_ref[...]
        hit_ref[...] = jnp.where(hi >= 0, hi, leaky * hi)


@jax.jit
def kernel(adj, user_emb, item_emb):
    n_users, n_items = adj.shape
    hd = user_emb.shape[1]
    bu = min(n_users, 2048)
    bi_sz = min(n_items, 2048)
    nbu = n_users // bu
    nbi = n_items // bi_sz

    p1 = functools.partial(_pass1_kernel, bu=bu, bi_sz=bi_sz,
                           nbu=nbu, nbi=nbi)
    su, tit, bm = pl.pallas_call(
        p1,
        grid=(nbu, nbi),
        in_specs=[
            pl.BlockSpec((bu, bi_sz), lambda i, j: (i, j)),
            pl.BlockSpec((hd, n_users), lambda i, j: (0, 0)),
            pl.BlockSpec((bi_sz, hd), lambda i, j: (j, 0)),
        ],
        out_specs=[
            pl.BlockSpec((hd, n_items), lambda i, j: (0, 0)),
            pl.BlockSpec((hd, n_users), lambda i, j: (0, 0)),
            pl.BlockSpec((bu // 32, bi_sz), lambda i, j: (i, j)),
        ],
        out_shape=[
            jax.ShapeDtypeStruct((hd, n_items), jnp.float32),
            jax.ShapeDtypeStruct((hd, n_users), jnp.float32),
            jax.ShapeDtypeStruct((n_users // 32, n_items), jnp.int32),
        ],
    )(adj, user_emb.T, item_emb)

    p2 = functools.partial(_pass2_kernel, bu=bu, bi_sz=bi_sz,
                           nbu=nbu, nbi=nbi, leaky=0.5)
    h_u, h_i_t = pl.pallas_call(
        p2,
        grid=(nbu, nbi),
        in_specs=[
            pl.BlockSpec((bu // 32, bi_sz), lambda i, j: (i, j)),
            pl.BlockSpec((hd, n_items), lambda i, j: (0, 0)),
            pl.BlockSpec((hd, n_users), lambda i, j: (0, 0)),
        ],
        out_specs=[
            pl.BlockSpec((bu, hd), lambda i, j: (i, 0)),
            pl.BlockSpec((hd, n_items), lambda i, j: (0, 0)),
        ],
        out_shape=[
            jax.ShapeDtypeStruct((n_users, hd), jnp.float32),
            jax.ShapeDtypeStruct((hd, n_items), jnp.float32),
        ],
        scratch_shapes=[
            pltpu.VMEM((hd, n_items), jnp.float32),
        ],
    )(bm, su, tit)

    user_all = jnp.concatenate([user_emb, h_u, h_u], axis=1)
    h_i = h_i_t.T
    item_all = jnp.concatenate([item_emb, h_i, h_i], axis=1)
    return (user_all, item_all)


# speedup vs baseline: 1.0074x; 1.0047x over previous
import functools

import jax
import jax.numpy as jnp
from jax.experimental import pallas as pl
from jax.experimental.pallas import tpu as pltpu

_MM = (((1,), (0,)), ((), ()))
_CH = 512


def _pass1_kernel(adj_ref, ut_ref, i_ref, su_ref, tit_ref, bm_ref,
                  *, bu, bi_sz, nbu, nbi):
    bi = pl.program_id(0)
    bj = pl.program_id(1)
    nch = bu // _CH

    @pl.when((bi == 0) & (bj == 0))
    def _init():
        su_ref[...] = jnp.zeros_like(su_ref)
        tit_ref[...] = jnp.zeros_like(tit_ref)

    ib = i_ref[...].astype(jnp.bfloat16)
    contrib = None
    su_acc = None
    for k in range(nch):
        sl = pl.ds(k * _CH, _CH)
        a_f32 = adj_ref[sl, :]
        ab = a_f32.astype(jnp.bfloat16)
        su_part = jax.lax.dot_general(
            ut_ref[:, pl.ds(bi * bu + k * _CH, _CH)].astype(jnp.bfloat16),
            ab, _MM, preferred_element_type=jnp.float32)
        su_acc = su_part if su_acc is None else su_acc + su_part
        if k == nch - 1:
            su_ref[:, pl.ds(bj * bi_sz, bi_sz)] += su_acc
        part = jax.lax.dot_general(
            ab, ib,
            _MM, preferred_element_type=jnp.float32)
        bits = a_f32.reshape(32, 16, bi_sz).astype(jnp.int32)
        sh = jax.lax.broadcasted_iota(jnp.int32, (32, 16, bi_sz), 0)
        bm_ref[pl.ds(k * 16, 16), :] = jnp.sum(
            jax.lax.shift_left(bits, sh), axis=0)
        contrib = part if contrib is None else jnp.concatenate(
            [contrib, part], axis=0)
        if k == nch - 1:
            tit_ref[:, pl.ds(bi * bu, bu)] += contrib.T


def _pass2_kernel(bm_ref, su_ref, tit_ref, hu_ref, hit_ref, shi_ref,
                  *, bu, bi_sz, nbu, nbi, leaky):
    bi = pl.program_id(0)
    bj = pl.program_id(1)
    nch = bu // _CH

    @pl.when((bi == 0) & (bj == 0))
    def _zshi():
        shi_ref[...] = jnp.zeros_like(shi_ref)

    @pl.when(bj == 0)
    def _zhu():
        hu_ref[...] = jnp.zeros_like(hu_ref)

    tu_blk = jnp.concatenate(
        [su_ref[:, pl.ds(bj * bi_sz + k * _CH, _CH)].astype(jnp.bfloat16).T
         for k in range(bi_sz // _CH)], axis=0)

    shi_acc = None
    for k in range(nch):
        words = bm_ref[pl.ds(k * 16, 16), :]
        ab = jnp.concatenate(
            [(jax.lax.shift_right_logical(words, m) & 1)
             for m in range(32)], axis=0).astype(jnp.bfloat16)
        hu_ref[pl.ds(k * _CH, _CH), :] += jax.lax.dot_general(
            ab, tu_blk, _MM, preferred_element_type=jnp.float32)
        shi_part = jax.lax.dot_general(
            tit_ref[:, pl.ds(bi * bu + k * _CH, _CH)].astype(jnp.bfloat16),
            ab, _MM, preferred_element_type=jnp.float32)
        shi_acc = shi_part if shi_acc is None else shi_acc + shi_part
        if k == nch - 1:
            shi_ref[:, pl.ds(bj * bi_sz, bi_sz)] += shi_acc

    @pl.when(bj == nbi - 1)
    def _act_u():
        huv = hu_ref[...]
        hu_ref[...] = jnp.where(huv >= 0, huv, leaky * huv)

    @pl.when((bi == nbu - 1) & (bj == nbi - 1))
    def _act_i():
        hi = shi_ref[...]
        hit_ref[...] = jnp.where(hi >= 0, hi, leaky * hi)


@jax.jit
def kernel(adj, user_emb, item_emb):
    n_users, n_items = adj.shape
    hd = user_emb.shape[1]
    bu = min(n_users, 2048)
    bi_sz = min(n_items, 2048)
    nbu = n_users // bu
    nbi = n_items // bi_sz

    p1 = functools.partial(_pass1_kernel, bu=bu, bi_sz=bi_sz,
                           nbu=nbu, nbi=nbi)
    su, tit, bm = pl.pallas_call(
        p1,
        grid=(nbu, nbi),
        in_specs=[
            pl.BlockSpec((bu, bi_sz), lambda i, j: (i, j)),
            pl.BlockSpec((hd, n_users), lambda i, j: (0, 0)),
            pl.BlockSpec((bi_sz, hd), lambda i, j: (j, 0)),
        ],
        out_specs=[
            pl.BlockSpec((hd, n_items), lambda i, j: (0, 0)),
            pl.BlockSpec((hd, n_users), lambda i, j: (0, 0)),
            pl.BlockSpec((bu // 32, bi_sz), lambda i, j: (i, j)),
        ],
        out_shape=[
            jax.ShapeDtypeStruct((hd, n_items), jnp.float32),
            jax.ShapeDtypeStruct((hd, n_users), jnp.float32),
            jax.ShapeDtypeStruct((n_users // 32, n_items), jnp.int32),
        ],
    )(adj, user_emb.T, item_emb)

    p2 = functools.partial(_pass2_kernel, bu=bu, bi_sz=bi_sz,
                           nbu=nbu, nbi=nbi, leaky=0.5)
    h_u, h_i_t = pl.pallas_call(
        p2,
        grid=(nbu, nbi),
        in_specs=[
            pl.BlockSpec((bu // 32, bi_sz), lambda i, j: (i, j)),
            pl.BlockSpec((hd, n_items), lambda i, j: (0, 0)),
            pl.BlockSpec((hd, n_users), lambda i, j: (0, 0)),
        ],
        out_specs=[
            pl.BlockSpec((bu, hd), lambda i, j: (i, 0)),
            pl.BlockSpec((hd, n_items), lambda i, j: (0, 0)),
        ],
        out_shape=[
            jax.ShapeDtypeStruct((n_users, hd), jnp.float32),
            jax.ShapeDtypeStruct((hd, n_items), jnp.float32),
        ],
        scratch_shapes=[
            pltpu.VMEM((hd, n_items), jnp.float32),
        ],
    )(bm, su, tit)

    user_all = jnp.concatenate([user_emb, h_u, h_u], axis=1)
    h_i = h_i_t.T
    item_all = jnp.concatenate([item_emb, h_i, h_i], axis=1)
    return (user_all, item_all)
